# baseline (speedup 1.0000x reference)
import functools

import jax
import jax.numpy as jnp
from jax.experimental import pallas as pl
from jax.experimental.pallas import tpu as pltpu

NUM_BANDS = 8192
K_BANDS = 2048
ROW_BLOCK = 128


def _gating_kernel(x_ref, w_ref, b_ref, out_ref, mask_out_ref, mask_scr):
    @pl.when(pl.program_id(0) == 0)
    def _compute_mask():
        w = w_ref[...]
        bits = jax.lax.bitcast_convert_type(w, jnp.int32) & jnp.int32(0x7FFFFFFF)
        thr = jnp.int32(0)
        for bit in range(30, -1, -1):
            cand = thr | jnp.int32(1 << bit)
            cnt = jnp.sum((bits >= cand).astype(jnp.int32))
            thr = jnp.where(cnt >= K_BANDS, cand, thr)
        mask = (bits >= thr).astype(jnp.float32)
        mask_out_ref[...] = mask
        mask_scr[...] = mask

    mask = mask_scr[...]
    w = w_ref[...]
    b = b_ref[...]
    out_ref[...] = jnp.where(mask > 0.0, x_ref[...] * w + b, b)


@jax.jit
def kernel(x, weights, bias):
    batch, num_bands = x.shape
    w2 = weights.reshape(1, num_bands)
    b2 = bias.reshape(1, num_bands)
    grid = (batch // ROW_BLOCK,)
    out, mask = pl.pallas_call(
        _gating_kernel,
        grid=grid,
        in_specs=[
            pl.BlockSpec((ROW_BLOCK, num_bands), lambda i: (i, 0)),
            pl.BlockSpec((1, num_bands), lambda i: (0, 0)),
            pl.BlockSpec((1, num_bands), lambda i: (0, 0)),
        ],
        out_specs=[
            pl.BlockSpec((ROW_BLOCK, num_bands), lambda i: (i, 0)),
            pl.BlockSpec((1, num_bands), lambda i: (0, 0)),
        ],
        out_shape=[
            jax.ShapeDtypeStruct((batch, num_bands), jnp.float32),
            jax.ShapeDtypeStruct((1, num_bands), jnp.float32),
        ],
        scratch_shapes=[pltpu.VMEM((1, num_bands), jnp.float32)],
    )(x, w2, b2)
    return out, mask.reshape(num_bands)


# speedup vs baseline: 1.0268x; 1.0268x over previous
import functools

import jax
import jax.numpy as jnp
from jax.experimental import pallas as pl
from jax.experimental.pallas import tpu as pltpu

NUM_BANDS = 8192
K_BANDS = 2048
ROW_BLOCK = 256


def _gating_kernel(x_ref, w_ref, b_ref, out_ref, mask_out_ref, mask_scr):
    @pl.when(pl.program_id(0) == 0)
    def _compute_mask():
        w = w_ref[...]
        bits = jax.lax.bitcast_convert_type(w, jnp.int32) & jnp.int32(0x7FFFFFFF)
        thr = jnp.int32(0)
        for bit in range(30, -1, -1):
            cand = thr | jnp.int32(1 << bit)
            cnt = jnp.sum((bits >= cand).astype(jnp.int32))
            thr = jnp.where(cnt >= K_BANDS, cand, thr)
        mask = (bits >= thr).astype(jnp.float32)
        mask_out_ref[...] = mask
        mask_scr[...] = w * mask

    out_ref[...] = x_ref[...] * mask_scr[...] + b_ref[...]


@jax.jit
def kernel(x, weights, bias):
    batch, num_bands = x.shape
    w2 = weights.reshape(1, num_bands)
    b2 = bias.reshape(1, num_bands)
    grid = (batch // ROW_BLOCK,)
    out, mask = pl.pallas_call(
        _gating_kernel,
        grid=grid,
        in_specs=[
            pl.BlockSpec((ROW_BLOCK, num_bands), lambda i: (i, 0)),
            pl.BlockSpec((1, num_bands), lambda i: (0, 0)),
            pl.BlockSpec((1, num_bands), lambda i: (0, 0)),
        ],
        out_specs=[
            pl.BlockSpec((ROW_BLOCK, num_bands), lambda i: (i, 0)),
            pl.BlockSpec((1, num_bands), lambda i: (0, 0)),
        ],
        out_shape=[
            jax.ShapeDtypeStruct((batch, num_bands), jnp.float32),
            jax.ShapeDtypeStruct((1, num_bands), jnp.float32),
        ],
        scratch_shapes=[pltpu.VMEM((1, num_bands), jnp.float32)],
        compiler_params=pltpu.CompilerParams(
            vmem_limit_bytes=128 * 1024 * 1024,
        ),
    )(x, w2, b2)
    return out, mask.reshape(num_bands)


# speedup vs baseline: 1.0604x; 1.0327x over previous
import functools

import jax
import jax.numpy as jnp
from jax.experimental import pallas as pl
from jax.experimental.pallas import tpu as pltpu

NUM_BANDS = 8192
K_BANDS = 2048
ROW_BLOCK = 448


def _gating_kernel(x_ref, w_ref, b_ref, out_ref, mask_out_ref, mask_scr):
    @pl.when(pl.program_id(0) == 0)
    def _compute_mask():
        w = w_ref[...]
        bits = jax.lax.bitcast_convert_type(w, jnp.int32) & jnp.int32(0x7FFFFFFF)
        thr = jnp.int32(0)
        for bit in range(30, -1, -1):
            cand = thr | jnp.int32(1 << bit)
            cnt = jnp.sum((bits >= cand).astype(jnp.int32))
            thr = jnp.where(cnt >= K_BANDS, cand, thr)
        mask = (bits >= thr).astype(jnp.float32)
        mask_out_ref[...] = mask
        mask_scr[...] = w * mask

    out_ref[...] = x_ref[...] * mask_scr[...] + b_ref[...]


@jax.jit
def kernel(x, weights, bias):
    batch, num_bands = x.shape
    w2 = weights.reshape(1, num_bands)
    b2 = bias.reshape(1, num_bands)
    grid = (pl.cdiv(batch, ROW_BLOCK),)
    out, mask = pl.pallas_call(
        _gating_kernel,
        grid=grid,
        in_specs=[
            pl.BlockSpec((ROW_BLOCK, num_bands), lambda i: (i, 0)),
            pl.BlockSpec((1, num_bands), lambda i: (0, 0)),
            pl.BlockSpec((1, num_bands), lambda i: (0, 0)),
        ],
        out_specs=[
            pl.BlockSpec((ROW_BLOCK, num_bands), lambda i: (i, 0)),
            pl.BlockSpec((1, num_bands), lambda i: (0, 0)),
        ],
        out_shape=[
            jax.ShapeDtypeStruct((batch, num_bands), jnp.float32),
            jax.ShapeDtypeStruct((1, num_bands), jnp.float32),
        ],
        scratch_shapes=[pltpu.VMEM((1, num_bands), jnp.float32)],
        compiler_params=pltpu.CompilerParams(
            vmem_limit_bytes=128 * 1024 * 1024,
        ),
    )(x, w2, b2)
    return out, mask.reshape(num_bands)


# speedup vs baseline: 1.0632x; 1.0027x over previous
import functools

import jax
import jax.numpy as jnp
from jax.experimental import pallas as pl
from jax.experimental.pallas import tpu as pltpu

NUM_BANDS = 8192
K_BANDS = 2048
ROW_BLOCK = 480


def _gating_kernel(x_ref, w_ref, b_ref, out_ref, mask_out_ref, mask_scr):
    @pl.when(pl.program_id(0) == 0)
    def _compute_mask():
        w = w_ref[...]
        bits = jax.lax.bitcast_convert_type(w, jnp.int32) & jnp.int32(0x7FFFFFFF)
        thr = jnp.int32(0)
        for bit in range(30, -1, -1):
            cand = thr | jnp.int32(1 << bit)
            cnt = jnp.sum((bits >= cand).astype(jnp.int32))
            thr = jnp.where(cnt >= K_BANDS, cand, thr)
        mask = (bits >= thr).astype(jnp.float32)
        mask_out_ref[...] = mask
        mask_scr[...] = w * mask

    out_ref[...] = x_ref[...] * mask_scr[...] + b_ref[...]


@jax.jit
def kernel(x, weights, bias):
    batch, num_bands = x.shape
    w2 = weights.reshape(1, num_bands)
    b2 = bias.reshape(1, num_bands)
    grid = (pl.cdiv(batch, ROW_BLOCK),)
    out, mask = pl.pallas_call(
        _gating_kernel,
        grid=grid,
        in_specs=[
            pl.BlockSpec((ROW_BLOCK, num_bands), lambda i: (i, 0)),
            pl.BlockSpec((1, num_bands), lambda i: (0, 0)),
            pl.BlockSpec((1, num_bands), lambda i: (0, 0)),
        ],
        out_specs=[
            pl.BlockSpec((ROW_BLOCK, num_bands), lambda i: (i, 0)),
            pl.BlockSpec((1, num_bands), lambda i: (0, 0)),
        ],
        out_shape=[
            jax.ShapeDtypeStruct((batch, num_bands), jnp.float32),
            jax.ShapeDtypeStruct((1, num_bands), jnp.float32),
        ],
        scratch_shapes=[pltpu.VMEM((1, num_bands), jnp.float32)],
        compiler_params=pltpu.CompilerParams(
            vmem_limit_bytes=128 * 1024 * 1024,
        ),
    )(x, w2, b2)
    return out, mask.reshape(num_bands)
